# Initial kernel scaffold; baseline (speedup 1.0000x reference)
import jax
import jax.numpy as jnp
from jax.experimental import pallas as pl


def kernel(x, w, gamma, beta):
    raise NotImplementedError("write your pallas kernel here")


# speedup vs baseline: 1.9189x; 1.9189x over previous
import functools

import jax
import jax.numpy as jnp
from jax import lax
from jax.experimental import pallas as pl
from jax.experimental.pallas import tpu as pltpu


def _branches(p_ref, w_ref, c_in):
    p = p_ref[...]
    p_conv = jnp.where(p == -jnp.inf, jnp.zeros((), p.dtype), p)
    main = jnp.dot(w_ref[...], p_conv,
                   preferred_element_type=jnp.float32)
    p32 = p.astype(jnp.float32)
    slabs = [p32[t * c_in:(t + 1) * c_in, :] for t in range(9)]
    while len(slabs) > 1:
        nxt = [jnp.maximum(slabs[i], slabs[i + 1])
               for i in range(0, len(slabs) - 1, 2)]
        if len(slabs) % 2:
            nxt.append(slabs[-1])
        slabs = nxt
    return main, slabs[0]


def _stats_kernel(p_ref, w_ref, stats_ref, *, c_in):
    main, ext = _branches(p_ref, w_ref, c_in)

    @pl.when(pl.program_id(1) == 0)
    def _():
        stats_ref[...] = jnp.zeros_like(stats_ref)

    s1 = jnp.concatenate([jnp.sum(main, axis=1, keepdims=True),
                          jnp.sum(ext, axis=1, keepdims=True)], axis=0)
    s2 = jnp.concatenate([jnp.sum(main * main, axis=1, keepdims=True),
                          jnp.sum(ext * ext, axis=1, keepdims=True)], axis=0)
    stats_ref[...] += jnp.concatenate([s1, s2], axis=1)


def _apply_kernel(p_ref, w_ref, stats_ref, gamma_ref, beta_ref, out_ref,
                  *, c_in, inv_count, eps):
    main, ext = _branches(p_ref, w_ref, c_in)
    pre = jnp.concatenate([main, ext], axis=0)
    st = jnp.sum(stats_ref[...], axis=0)
    mean = st[:, 0:1] * inv_count
    var = st[:, 1:2] * inv_count - mean * mean
    scale = gamma_ref[...] * lax.rsqrt(var + eps)
    shift = beta_ref[...] - mean * scale
    out_ref[...] = jnp.maximum(pre * scale + shift, 0.0)


def _pick_tile(s, max_tile=8192):
    best = None
    t = 128
    while t <= min(s, max_tile):
        if s % t == 0:
            best = t
        t += 128
    return best if best is not None else s


def kernel(x, w, gamma, beta, eps=1e-5):
    N, C_in, H, W = x.shape
    C_conv = w.shape[0]
    C_out = C_conv + C_in
    Ho = (H - 1) // 2 + 1
    Wo = (W - 1) // 2 + 1
    S = Ho * Wo
    L = _pick_tile(S)
    T = S // L
    R = N * S

    xb = x.astype(jnp.bfloat16)
    x_pad = jnp.pad(xb, ((0, 0), (0, 0), (1, 1), (1, 1)),
                    constant_values=-jnp.inf)
    taps = [x_pad[:, :, dh:dh + 2 * Ho - 1:2, dw:dw + 2 * Wo - 1:2]
            for dh in range(3) for dw in range(3)]
    patches = jnp.stack(taps, axis=1).reshape(N, 9 * C_in, S)

    w_mat = jnp.transpose(w, (0, 2, 3, 1)).reshape(C_conv, 9 * C_in)
    w_mat = w_mat.astype(jnp.bfloat16)
    gamma2 = gamma.reshape(C_out, 1).astype(jnp.float32)
    beta2 = beta.reshape(C_out, 1).astype(jnp.float32)

    p_spec = pl.BlockSpec((None, 9 * C_in, L), lambda n, t: (n, 0, t))
    w_spec = pl.BlockSpec((C_conv, 9 * C_in), lambda n, t: (0, 0))

    stats = pl.pallas_call(
        functools.partial(_stats_kernel, c_in=C_in),
        out_shape=jax.ShapeDtypeStruct((N, C_out, 2), jnp.float32),
        grid=(N, T),
        in_specs=[p_spec, w_spec],
        out_specs=pl.BlockSpec((None, C_out, 2), lambda n, t: (n, 0, 0)),
        compiler_params=pltpu.CompilerParams(
            dimension_semantics=("parallel", "arbitrary")),
    )(patches, w_mat)

    out = pl.pallas_call(
        functools.partial(_apply_kernel, c_in=C_in,
                          inv_count=1.0 / R, eps=eps),
        out_shape=jax.ShapeDtypeStruct((N, C_out, S), jnp.float32),
        grid=(N, T),
        in_specs=[p_spec, w_spec,
                  pl.BlockSpec((N, C_out, 2), lambda n, t: (0, 0, 0)),
                  pl.BlockSpec((C_out, 1), lambda n, t: (0, 0)),
                  pl.BlockSpec((C_out, 1), lambda n, t: (0, 0))],
        out_specs=pl.BlockSpec((None, C_out, L), lambda n, t: (n, 0, t)),
        compiler_params=pltpu.CompilerParams(
            dimension_semantics=("parallel", "parallel")),
    )(patches, w_mat, stats, gamma2, beta2)

    return out.reshape(N, C_out, Ho, Wo)


# speedup vs baseline: 41.5666x; 21.6614x over previous
import functools

import numpy as np

import jax
import jax.numpy as jnp
from jax import lax
from jax.experimental import pallas as pl
from jax.experimental.pallas import tpu as pltpu

_NEG_INF = float("-inf")
_LANES = 128


def _sel_matrix(w_in, w_out):
    s = np.zeros((w_in, 3 * w_out), dtype=np.float32)
    for c in range(w_out):
        u = 2 * c
        if u < w_in:
            s[u, c] = 1.0
        if u + 1 < w_in:
            s[u + 1, w_out + c] = 1.0
        if u - 1 >= 0:
            s[u - 1, 2 * w_out + c] = 1.0
    return jnp.asarray(s, dtype=jnp.bfloat16)


def _stats_kernel(x_ref, s_ref, w_ref, stats_ref, pre_ref, scratch,
                  *, h, w, ho, wo, ch):
    c_in = x_ref.shape[1]
    c_conv = w_ref.shape[0]
    c_out = c_conv + c_in
    n_half = wo // _LANES
    xb = x_ref[0].reshape(c_in * h, w).astype(jnp.bfloat16)
    planes = jnp.dot(xb, s_ref[...],
                     preferred_element_type=jnp.float32)
    for j in range(3 * n_half):
        scratch[j] = planes[:, j * _LANES:(j + 1) * _LANES]

    iota = lax.broadcasted_iota(jnp.int32, (ch, _LANES), 1)
    groups = [range(0, c_conv // 2), range(c_conv // 2, c_conv)]
    acc1 = [jnp.zeros((c_out, _LANES), jnp.float32) for _ in range(n_half)]
    acc2 = [jnp.zeros((c_out, _LANES), jnp.float32) for _ in range(n_half)]

    for r0 in range(0, ho, ch):
        for jh in range(n_half):
            accs = [None] * c_conv
            exts = [None] * c_in
            for gi, group in enumerate(groups):
                last = gi == len(groups) - 1
                for ci in range(c_in):
                    subs = [None] * c_conv
                    for dh in range(3):
                        for dw in range(3):
                            reg = (2 if dw == 0 else dw - 1) * n_half + jh
                            base = ci * h + 2 * r0 + dh - 1
                            if dh == 0 and r0 == 0:
                                body = scratch[
                                    reg, pl.Slice(ci * h + 1, ch - 1, 2),
                                    :].astype(jnp.bfloat16)
                                conv_tap = jnp.concatenate(
                                    [jnp.zeros((1, _LANES), jnp.bfloat16),
                                     body], axis=0)
                            else:
                                conv_tap = scratch[
                                    reg, pl.Slice(base, ch, 2),
                                    :].astype(jnp.bfloat16)
                            k = ci * 9 + dh * 3 + dw
                            for co in group:
                                t = w_ref[co, k] * conv_tap
                                subs[co] = (t if subs[co] is None
                                            else subs[co] + t)
                            if last:
                                pool_tap = conv_tap
                                if dh == 0 and r0 == 0:
                                    pool_tap = jnp.concatenate(
                                        [jnp.full((1, _LANES), _NEG_INF,
                                                  jnp.bfloat16), body], axis=0)
                                if dw == 0 and jh == 0:
                                    pool_tap = jnp.where(
                                        iota == 0,
                                        jnp.bfloat16(_NEG_INF), pool_tap)
                                exts[ci] = (pool_tap if exts[ci] is None
                                            else jnp.maximum(exts[ci],
                                                             pool_tap))
                    for co in group:
                        accs[co] = (subs[co] if accs[co] is None
                                    else accs[co] + subs[co])
            pre = jnp.concatenate([jnp.stack(accs, axis=0),
                                   jnp.stack(exts, axis=0)], axis=0)
            pre_ref[:, r0:r0 + ch, jh * _LANES:(jh + 1) * _LANES] = pre
            pre32 = pre.astype(jnp.float32)
            acc1[jh] = acc1[jh] + jnp.sum(pre32, axis=1)
            acc2[jh] = acc2[jh] + jnp.sum(pre32 * pre32, axis=1)
    stats_ref[...] = jnp.concatenate(acc1 + acc2, axis=1)


def _apply_kernel(pre_ref, stats_ref, gamma_ref, beta_ref, out_ref,
                  *, wo, inv_count, eps):
    st = jnp.sum(stats_ref[...], axis=0)
    s1 = jnp.sum(st[:, :wo], axis=1, keepdims=True)
    s2 = jnp.sum(st[:, wo:], axis=1, keepdims=True)
    mean = s1 * inv_count
    var = s2 * inv_count - mean * mean
    scale = gamma_ref[...] * lax.rsqrt(var + eps)
    shift = beta_ref[...] - mean * scale
    scale = scale.reshape(-1, 1, 1)
    shift = shift.reshape(-1, 1, 1)
    pre = pre_ref[...].astype(jnp.float32)
    out_ref[...] = jnp.maximum(pre * scale + shift, 0.0)


def kernel(x, w, gamma, beta, eps=1e-5):
    N, C_in, H, W = x.shape
    C_conv = w.shape[0]
    C_out = C_conv + C_in
    Ho = (H - 1) // 2 + 1
    Wo = (W - 1) // 2 + 1
    R = N * Ho * Wo
    CH = 64 if Ho % 64 == 0 else Ho

    sel = _sel_matrix(W, Wo)
    w_mat = w.reshape(C_conv, 9 * C_in).astype(jnp.bfloat16)
    gamma2 = gamma.reshape(C_out, 1).astype(jnp.float32)
    beta2 = beta.reshape(C_out, 1).astype(jnp.float32)

    stats, pre = pl.pallas_call(
        functools.partial(_stats_kernel, h=H, w=W, ho=Ho, wo=Wo, ch=CH),
        out_shape=(jax.ShapeDtypeStruct((N, C_out, 2 * Wo), jnp.float32),
                   jax.ShapeDtypeStruct((N, C_out, Ho, Wo), jnp.bfloat16)),
        grid=(N,),
        in_specs=[pl.BlockSpec((1, C_in, H, W), lambda n: (n, 0, 0, 0)),
                  pl.BlockSpec((W, 3 * Wo), lambda n: (0, 0)),
                  pl.BlockSpec(memory_space=pltpu.SMEM)],
        out_specs=(pl.BlockSpec((None, C_out, 2 * Wo), lambda n: (n, 0, 0)),
                   pl.BlockSpec((None, C_out, Ho, Wo),
                                lambda n: (n, 0, 0, 0))),
        scratch_shapes=[pltpu.VMEM((3 * (Wo // _LANES), C_in * H, _LANES),
                                   jnp.float32)],
        compiler_params=pltpu.CompilerParams(
            dimension_semantics=("parallel",)),
    )(x, sel, w_mat)

    out = pl.pallas_call(
        functools.partial(_apply_kernel, wo=Wo, inv_count=1.0 / R, eps=eps),
        out_shape=jax.ShapeDtypeStruct((N, C_out, Ho, Wo), jnp.float32),
        grid=(N,),
        in_specs=[pl.BlockSpec((None, C_out, Ho, Wo), lambda n: (n, 0, 0, 0)),
                  pl.BlockSpec((N, C_out, 2 * Wo), lambda n: (0, 0, 0)),
                  pl.BlockSpec((C_out, 1), lambda n: (0, 0)),
                  pl.BlockSpec((C_out, 1), lambda n: (0, 0))],
        out_specs=pl.BlockSpec((None, C_out, Ho, Wo), lambda n: (n, 0, 0, 0)),
        compiler_params=pltpu.CompilerParams(
            dimension_semantics=("parallel",)),
    )(pre, stats, gamma2, beta2)

    return out


# speedup vs baseline: 43.3663x; 1.0433x over previous
import functools

import numpy as np

import jax
import jax.numpy as jnp
from jax import lax
from jax.experimental import pallas as pl
from jax.experimental.pallas import tpu as pltpu

_NEG_INF = float("-inf")
_LANES = 128


def _sel_matrix(w_in, w_out):
    s = np.zeros((w_in, 3 * w_out), dtype=np.float32)
    for c in range(w_out):
        u = 2 * c
        if u < w_in:
            s[u, c] = 1.0
        if u + 1 < w_in:
            s[u + 1, w_out + c] = 1.0
        if u - 1 >= 0:
            s[u - 1, 2 * w_out + c] = 1.0
    return jnp.asarray(s, dtype=jnp.bfloat16)


def _stats_kernel(x_ref, s_ref, w_ref, stats_ref, pre_ref, scratch,
                  *, h, w, ho, wo, ch):
    c_in = x_ref.shape[1]
    c_conv = w_ref.shape[0]
    c_out = c_conv + c_in
    n_half = wo // _LANES
    xb = x_ref[0].reshape(c_in * h, w).astype(jnp.bfloat16)
    planes = jnp.dot(xb, s_ref[...],
                     preferred_element_type=jnp.float32)
    for j in range(3 * n_half):
        scratch[j] = planes[:, j * _LANES:(j + 1) * _LANES]

    iota = lax.broadcasted_iota(jnp.int32, (ch, _LANES), 1)
    g3 = (c_conv + 2) // 3
    groups = [range(0, g3), range(g3, 2 * g3), range(2 * g3, c_conv)]
    acc1 = [jnp.zeros((c_out, _LANES), jnp.float32) for _ in range(n_half)]
    acc2 = [jnp.zeros((c_out, _LANES), jnp.float32) for _ in range(n_half)]

    for r0 in range(0, ho, ch):
        for jh in range(n_half):
            accs = [None] * c_conv
            exts = [None] * c_in
            for gi, group in enumerate(groups):
                last = gi == len(groups) - 1
                for ci in range(c_in):
                    subs = [None] * c_conv
                    for dh in range(3):
                        for dw in range(3):
                            reg = (2 if dw == 0 else dw - 1) * n_half + jh
                            base = ci * h + 2 * r0 + dh - 1
                            if dh == 0 and r0 == 0:
                                body = scratch[
                                    reg, pl.Slice(ci * h + 1, ch - 1, 2),
                                    :].astype(jnp.bfloat16)
                                conv_tap = jnp.concatenate(
                                    [jnp.zeros((1, _LANES), jnp.bfloat16),
                                     body], axis=0)
                            else:
                                conv_tap = scratch[
                                    reg, pl.Slice(base, ch, 2),
                                    :].astype(jnp.bfloat16)
                            k = ci * 9 + dh * 3 + dw
                            for co in group:
                                t = w_ref[co, k] * conv_tap
                                subs[co] = (t if subs[co] is None
                                            else subs[co] + t)
                            if last:
                                pool_tap = conv_tap
                                if dh == 0 and r0 == 0:
                                    pool_tap = jnp.concatenate(
                                        [jnp.full((1, _LANES), _NEG_INF,
                                                  jnp.bfloat16), body], axis=0)
                                if dw == 0 and jh == 0:
                                    pool_tap = jnp.where(
                                        iota == 0,
                                        jnp.bfloat16(_NEG_INF), pool_tap)
                                exts[ci] = (pool_tap if exts[ci] is None
                                            else jnp.maximum(exts[ci],
                                                             pool_tap))
                    for co in group:
                        accs[co] = (subs[co] if accs[co] is None
                                    else accs[co] + subs[co])
            pre = jnp.concatenate([jnp.stack(accs, axis=0),
                                   jnp.stack(exts, axis=0)], axis=0)
            pre_ref[:, r0:r0 + ch, jh * _LANES:(jh + 1) * _LANES] = pre

            def _tree32(v):
                while v.shape[1] > 16 and v.shape[1] % 2 == 0:
                    half = v.shape[1] // 2
                    v = v[:, :half] + v[:, half:]
                return jnp.sum(v.astype(jnp.float32), axis=1)

            sq = pre * pre
            acc1[jh] = acc1[jh] + _tree32(pre)
            acc2[jh] = acc2[jh] + _tree32(sq)
    stats_ref[...] = jnp.concatenate(acc1 + acc2, axis=1)


def _apply_kernel(pre_ref, stats_ref, gamma_ref, beta_ref, out_ref,
                  *, wo, inv_count, eps):
    st = jnp.sum(stats_ref[...], axis=0)
    s1 = jnp.sum(st[:, :wo], axis=1, keepdims=True)
    s2 = jnp.sum(st[:, wo:], axis=1, keepdims=True)
    mean = s1 * inv_count
    var = s2 * inv_count - mean * mean
    scale = gamma_ref[...] * lax.rsqrt(var + eps)
    shift = beta_ref[...] - mean * scale
    scale = scale.reshape(-1, 1, 1)
    shift = shift.reshape(-1, 1, 1)
    pre = pre_ref[...].astype(jnp.float32)
    out_ref[...] = jnp.maximum(pre * scale + shift, 0.0)


def kernel(x, w, gamma, beta, eps=1e-5):
    N, C_in, H, W = x.shape
    C_conv = w.shape[0]
    C_out = C_conv + C_in
    Ho = (H - 1) // 2 + 1
    Wo = (W - 1) // 2 + 1
    R = N * Ho * Wo
    CH = 256 if Ho % 256 == 0 else Ho

    sel = _sel_matrix(W, Wo)
    w_mat = w.reshape(C_conv, 9 * C_in).astype(jnp.bfloat16)
    gamma2 = gamma.reshape(C_out, 1).astype(jnp.float32)
    beta2 = beta.reshape(C_out, 1).astype(jnp.float32)

    stats, pre = pl.pallas_call(
        functools.partial(_stats_kernel, h=H, w=W, ho=Ho, wo=Wo, ch=CH),
        out_shape=(jax.ShapeDtypeStruct((N, C_out, 2 * Wo), jnp.float32),
                   jax.ShapeDtypeStruct((N, C_out, Ho, Wo), jnp.bfloat16)),
        grid=(N,),
        in_specs=[pl.BlockSpec((1, C_in, H, W), lambda n: (n, 0, 0, 0)),
                  pl.BlockSpec((W, 3 * Wo), lambda n: (0, 0)),
                  pl.BlockSpec(memory_space=pltpu.SMEM)],
        out_specs=(pl.BlockSpec((None, C_out, 2 * Wo), lambda n: (n, 0, 0)),
                   pl.BlockSpec((None, C_out, Ho, Wo),
                                lambda n: (n, 0, 0, 0))),
        scratch_shapes=[pltpu.VMEM((3 * (Wo // _LANES), C_in * H, _LANES),
                                   jnp.float32)],
        compiler_params=pltpu.CompilerParams(
            dimension_semantics=("parallel",)),
    )(x, sel, w_mat)

    out = pl.pallas_call(
        functools.partial(_apply_kernel, wo=Wo, inv_count=1.0 / R, eps=eps),
        out_shape=jax.ShapeDtypeStruct((N, C_out, Ho, Wo), jnp.float32),
        grid=(N,),
        in_specs=[pl.BlockSpec((None, C_out, Ho, Wo), lambda n: (n, 0, 0, 0)),
                  pl.BlockSpec((N, C_out, 2 * Wo), lambda n: (0, 0, 0)),
                  pl.BlockSpec((C_out, 1), lambda n: (0, 0)),
                  pl.BlockSpec((C_out, 1), lambda n: (0, 0))],
        out_specs=pl.BlockSpec((None, C_out, Ho, Wo), lambda n: (n, 0, 0, 0)),
        compiler_params=pltpu.CompilerParams(
            dimension_semantics=("parallel",)),
    )(pre, stats, gamma2, beta2)

    return out
